# baseline (speedup 1.0000x reference)
import functools

import jax
import jax.numpy as jnp
from jax import lax
from jax.experimental import pallas as pl
from jax.experimental.pallas import tpu as pltpu

N = 10000
E = 160000
D = 256
H = 8
C = 32
K = 200


def _mm_kernel(x_ref, w_ref, o_ref):
    o_ref[...] = jnp.dot(x_ref[...], w_ref[...],
                         preferred_element_type=jnp.float32)


def _matmul(x, w):
    return pl.pallas_call(
        _mm_kernel,
        out_shape=jax.ShapeDtypeStruct((x.shape[0], w.shape[1]), jnp.float32),
    )(x, w)


def kernel(x, edge_index, W1, b1, Wg_root, Wg_rel, bg, W2, b2):
    src = edge_index[0]
    dst = edge_index[1]
    n = N
    si = jnp.arange(n, dtype=src.dtype)
    s = jnp.concatenate([src, si])
    d = jnp.concatenate([dst, si])
    deg = jax.ops.segment_sum(jnp.ones((s.shape[0],), x.dtype), d, num_segments=n)
    dinv = jnp.where(deg > 0, 1.0 / jnp.sqrt(jnp.where(deg > 0, deg, 1.0)), 0.0)
    norm = dinv[s] * dinv[d]
    h = _matmul(x, W1)
    out = jax.ops.segment_sum(h[s] * norm[:, None], d, num_segments=n)
    h = out + b1
    h = jax.nn.relu(h)
    agg = jax.ops.segment_sum(h[src], dst, num_segments=N)
    score = (h @ Wg_root + agg @ Wg_rel + bg).squeeze(-1)
    vals, perm = jax.lax.top_k(score, K)
    xp = h[perm] * jnp.tanh(vals)[:, None]
    node_new = jnp.full((N,), K, dtype=jnp.int32).at[perm].set(jnp.arange(K, dtype=jnp.int32))
    sn = node_new[src]
    dn = node_new[dst]
    mask = (sn < K) & (dn < K)
    w = mask.astype(xp.dtype)
    sc = jnp.where(mask, sn, K)
    dc = jnp.where(mask, dn, K)
    deg2 = jax.ops.segment_sum(w, dc, num_segments=K + 1)[:K] + 1.0
    dinv2 = 1.0 / jnp.sqrt(deg2)
    dinv_ext = jnp.concatenate([dinv2, jnp.zeros((1,), xp.dtype)])
    norm2 = dinv_ext[sc] * dinv_ext[dc] * w
    h2 = xp @ W2
    agg2 = jax.ops.segment_sum(h2[sc] * norm2[:, None], dc, num_segments=K + 1)[:K]
    out = agg2 + h2 * (dinv2 ** 2)[:, None] + b2
    return jax.nn.log_softmax(out, axis=1)


# speedup vs baseline: 56.3507x; 56.3507x over previous
import functools

import jax
import jax.numpy as jnp
from jax import lax
from jax.experimental import pallas as pl
from jax.experimental.pallas import tpu as pltpu
from jax.experimental.pallas import tpu_sc as plsc

N = 10000
E = 160000
D = 256
H = 8
C = 32
K = 200

NPAD = 10240
NC, NS, L = 2, 16, 16
NW = NC * NS

_SC_MESH = plsc.VectorSubcoreMesh(core_axis_name="c", subcore_axis_name="s")
_SC_PARAMS = pltpu.CompilerParams(needs_layout_passes=False)


def _zero_vmem(ref, nwords):
    z = jnp.zeros((L,), ref.dtype)

    def body(i, _):
        ref[pl.ds(i * L, L)] = z
        return 0

    lax.fori_loop(0, nwords // L, body, 0)


_EPW = E // NW
_EPW_PAD = _EPW + 128 - (_EPW % 128)
_ACC_PAD = NPAD + 128


@functools.partial(
    pl.kernel,
    out_type=jax.ShapeDtypeStruct((NW, NPAD), jnp.float32),
    mesh=_SC_MESH,
    scratch_types=[
        pltpu.VMEM((_EPW_PAD,), jnp.int32),
        pltpu.VMEM((_ACC_PAD,), jnp.float32),
    ],
    compiler_params=_SC_PARAMS,
)
def _sc_deg(dst_hbm, out_hbm, idx_v, acc_v):
    wid = lax.axis_index("s") * NC + lax.axis_index("c")
    base = wid * _EPW
    dummy = jnp.full((L,), NPAD, jnp.int32)
    _tail0 = (_EPW // L) * L

    def fill(i, _):
        idx_v[pl.ds(_tail0 + i * L, L)] = dummy
        return 0

    lax.fori_loop(0, (_EPW_PAD - _tail0) // L, fill, 0)
    pltpu.sync_copy(dst_hbm.at[pl.ds(base, _EPW)], idx_v.at[pl.ds(0, _EPW)])
    _zero_vmem(acc_v, _ACC_PAD)
    ones = jnp.full((L,), 1.0, jnp.float32)

    def body(i, _):
        idx = idx_v[pl.ds(i * L, L)]
        plsc.addupdate_scatter(acc_v, [idx], ones)
        return 0

    lax.fori_loop(0, _EPW_PAD // L, body, 0)
    pltpu.sync_copy(acc_v.at[pl.ds(0, NPAD)], out_hbm.at[wid])


_ECH = E // (NW // 2)


@functools.partial(
    pl.kernel,
    out_type=jax.ShapeDtypeStruct((NW, 4, NPAD), jnp.float32),
    mesh=_SC_MESH,
    scratch_types=[
        pltpu.VMEM((10240,), jnp.int32),
        pltpu.VMEM((10240,), jnp.int32),
        pltpu.VMEM((4, NPAD), jnp.float32),
        pltpu.VMEM((4, NPAD + 128), jnp.float32),
    ],
    compiler_params=_SC_PARAMS,
)
def _sc_msg(src_hbm, dst_hbm, g_hbm, out_hbm, src_v, dst_v, g4_v, acc_v):
    wid = lax.axis_index("s") * NC + lax.axis_index("c")
    g2 = wid % 2
    chunk = wid // 2
    base = chunk * _ECH
    pltpu.sync_copy(src_hbm.at[pl.ds(base, _ECH)], src_v.at[pl.ds(0, _ECH)])
    pltpu.sync_copy(dst_hbm.at[pl.ds(base, _ECH)], dst_v.at[pl.ds(0, _ECH)])
    pltpu.sync_copy(g_hbm.at[pl.ds(4 * g2, 4), :], g4_v)
    z = jnp.zeros((L,), jnp.float32)

    def zbody(i, _):
        acc_v[i // ((NPAD + 128) // L), pl.ds((i % ((NPAD + 128) // L)) * L, L)] = z
        return 0

    lax.fori_loop(0, 4 * ((NPAD + 128) // L), zbody, 0)

    fvecs = [jnp.full((L,), f, jnp.int32) for f in range(4)]

    def body(i, _):
        sv = src_v[pl.ds(i * L, L)]
        dv = dst_v[pl.ds(i * L, L)]
        for f in range(4):
            val = plsc.load_gather(g4_v, [fvecs[f], sv])
            plsc.addupdate_scatter(acc_v, [fvecs[f], dv], val)
        return 0

    lax.fori_loop(0, _ECH // L, body, 0)
    pltpu.sync_copy(acc_v.at[:, pl.ds(0, NPAD)], out_hbm.at[wid])


_A2 = 208


@functools.partial(
    pl.kernel,
    out_type=jax.ShapeDtypeStruct((NW, _A2, _A2), jnp.float32),
    mesh=_SC_MESH,
    scratch_types=[
        pltpu.VMEM((_EPW_PAD,), jnp.int32),
        pltpu.VMEM((_EPW_PAD,), jnp.int32),
        pltpu.VMEM((NPAD + 128,), jnp.int32),
        pltpu.VMEM((256,), jnp.int32),
        pltpu.VMEM((_A2, _A2), jnp.float32),
    ],
    compiler_params=_SC_PARAMS,
)
def _sc_conv2(src_hbm, dst_hbm, perm_hbm, out_hbm, src_v, dst_v, nn_v, perm_v,
              a2_v):
    wid = lax.axis_index("s") * NC + lax.axis_index("c")
    base = wid * _EPW
    dummy = jnp.full((L,), NPAD, jnp.int32)
    _tail0 = (_EPW // L) * L

    def fill(i, _):
        src_v[pl.ds(_tail0 + i * L, L)] = dummy
        dst_v[pl.ds(_tail0 + i * L, L)] = dummy
        return 0

    lax.fori_loop(0, (_EPW_PAD - _tail0) // L, fill, 0)
    pltpu.sync_copy(src_hbm.at[pl.ds(base, _EPW)], src_v.at[pl.ds(0, _EPW)])
    pltpu.sync_copy(dst_hbm.at[pl.ds(base, _EPW)], dst_v.at[pl.ds(0, _EPW)])
    pltpu.sync_copy(perm_hbm.at[:], perm_v)

    kvec = jnp.full((L,), K, jnp.int32)

    def nfill(i, _):
        nn_v[pl.ds(i * L, L)] = kvec
        return 0

    lax.fori_loop(0, (NPAD + 128) // L, nfill, 0)
    iota = lax.iota(jnp.int32, L)

    def pscatter(j, _):
        pv = perm_v[pl.ds(j * L, L)]
        plsc.store_scatter(nn_v, [pv], iota + j * L)
        return 0

    lax.fori_loop(0, 256 // L, pscatter, 0)

    z = jnp.zeros((L,), jnp.float32)

    def zbody(i, _):
        a2_v[i // (_A2 // L), pl.ds((i % (_A2 // L)) * L, L)] = z
        return 0

    lax.fori_loop(0, _A2 * (_A2 // L), zbody, 0)

    ones = jnp.full((L,), 1.0, jnp.float32)

    def body(i, _):
        sv = src_v[pl.ds(i * L, L)]
        dv = dst_v[pl.ds(i * L, L)]
        sn = plsc.load_gather(nn_v, [sv])
        dn = plsc.load_gather(nn_v, [dv])
        m = (sn < K) & (dn < K)
        snc = jnp.minimum(sn, K)
        dnc = jnp.minimum(dn, K)
        plsc.addupdate_scatter(a2_v, [dnc, snc], ones, mask=m)
        return 0

    lax.fori_loop(0, _EPW_PAD // L, body, 0)
    pltpu.sync_copy(a2_v, out_hbm.at[wid])


def _mm_kernel(x_ref, w_ref, o_ref):
    o_ref[...] = jnp.dot(x_ref[...], w_ref[...],
                         preferred_element_type=jnp.float32)


def _matmul(x, w):
    return pl.pallas_call(
        _mm_kernel,
        out_shape=jax.ShapeDtypeStruct((x.shape[0], w.shape[1]), jnp.float32),
    )(x, w)


def kernel(x, edge_index, W1, b1, Wg_root, Wg_rel, bg, W2, b2):
    src = edge_index[0]
    dst = edge_index[1]

    deg_parts = _sc_deg(dst)
    deg = deg_parts.sum(axis=0)[:N] + 1.0
    dinv = 1.0 / jnp.sqrt(deg)

    h0 = _matmul(x, W1)
    g = h0 * dinv[:, None]
    gT = jnp.zeros((H, NPAD), jnp.float32).at[:, :N].set(g.T)

    msg_parts = _sc_msg(src, dst, gT)
    msgT = msg_parts.reshape(NW // 2, 2, 4, NPAD).sum(axis=0).reshape(H, NPAD)
    m = msgT[:, :N].T
    h = jax.nn.relu(dinv[:, None] * (m + g) + b1)

    hT = jnp.zeros((H, NPAD), jnp.float32).at[:, :N].set(h.T)
    agg_parts = _sc_msg(src, dst, hT)
    aggT = agg_parts.reshape(NW // 2, 2, 4, NPAD).sum(axis=0).reshape(H, NPAD)
    agg = aggT[:, :N].T
    score = (h @ Wg_root + agg @ Wg_rel + bg).squeeze(-1)

    vals, perm = jax.lax.top_k(score, K)
    xp = h[perm] * jnp.tanh(vals)[:, None]

    perm256 = jnp.full((256,), NPAD, jnp.int32).at[:K].set(perm.astype(jnp.int32))
    a2_parts = _sc_conv2(src, dst, perm256)
    cnt = a2_parts.sum(axis=0)

    deg2 = cnt[:K, :K].sum(axis=1) + 1.0
    dinv2 = 1.0 / jnp.sqrt(deg2)
    a2n = cnt[:K, :K] * dinv2[:, None] * dinv2[None, :]
    h2 = xp @ W2
    agg2 = jnp.dot(a2n, h2, precision=lax.Precision.HIGHEST)
    out = agg2 + h2 * (dinv2 ** 2)[:, None] + b2
    return jax.nn.log_softmax(out, axis=1)
